# baseline (speedup 1.0000x reference)
import functools

import jax
import jax.numpy as jnp
from jax import lax
from jax.experimental import pallas as pl
from jax.experimental.pallas import tpu as pltpu
from jax.experimental.pallas import tpu_sc as plsc

NC = 2
NS = 16
NW = NC * NS
L = 16
B = 16384
D = 64
BPW = B // NW
CHUNK = 128
NCH = BPW // CHUNK

_mesh = plsc.VectorSubcoreMesh(core_axis_name="c", subcore_axis_name="s",
                               num_cores=NC, num_subcores=NS)


@functools.partial(
    pl.kernel,
    out_type=jax.ShapeDtypeStruct((B,), jnp.float32),
    mesh=_mesh,
    scratch_types=[
        pltpu.VMEM((NCH, CHUNK), jnp.int32),
        pltpu.VMEM((NCH, CHUNK), jnp.int32),
        pltpu.VMEM((BPW, D), jnp.float32),
        pltpu.VMEM((BPW, D), jnp.float32),
        pltpu.VMEM((BPW,), jnp.float32),
        pltpu.VMEM((BPW,), jnp.float32),
        pltpu.VMEM((BPW,), jnp.float32),
        pltpu.VMEM((L,), jnp.float32),
        pltpu.SemaphoreType.DMA,
    ],
    compiler_params=pltpu.CompilerParams(needs_layout_passes=False,
                                         use_tc_tiling_on_sc=False),
)
def _mf_kernel(user, item, uf, itf, ubias, ibias, gbias, out,
               uidx, iidx, urows, irows, ubv, ibv, outv, gbv, sem):
    wid = lax.axis_index("s") * NC + lax.axis_index("c")
    base = wid * BPW

    for j in range(NCH):
        pltpu.sync_copy(user.at[pl.ds(base + j * CHUNK, CHUNK)], uidx.at[j])
        pltpu.sync_copy(item.at[pl.ds(base + j * CHUNK, CHUNK)], iidx.at[j])
    pltpu.sync_copy(gbias.at[pl.ds(0, 1)], gbv.at[pl.ds(0, 1)])

    copies = []
    for j in range(NCH):
        dst = pl.ds(j * CHUNK, CHUNK)
        copies.append(pltpu.async_copy(uf.at[uidx.at[j]], urows.at[dst], sem))
        copies.append(pltpu.async_copy(itf.at[iidx.at[j]], irows.at[dst], sem))
        copies.append(pltpu.async_copy(ubias.at[uidx.at[j]], ubv.at[dst], sem))
        copies.append(pltpu.async_copy(ibias.at[iidx.at[j]], ibv.at[dst], sem))
    for c in copies:
        c.wait()

    gb = gbv[...][0]

    def group(g, carry):
        rid = lax.iota(jnp.int32, L) + g * L
        acc = ubv[pl.ds(g * L, L)] + ibv[pl.ds(g * L, L)] + gb
        for c in range(D):
            col = jnp.full((L,), c, jnp.int32)
            acc = acc + (plsc.load_gather(urows, [rid, col])
                         * plsc.load_gather(irows, [rid, col]))
        outv[pl.ds(g * L, L)] = acc
        return carry

    lax.fori_loop(0, BPW // L, group, 0)

    pltpu.sync_copy(outv, out.at[pl.ds(base, BPW)])


def kernel(user, item, user_factors, item_factors, user_biases, item_biases,
           global_bias):
    return _mf_kernel(user, item, user_factors, item_factors,
                      user_biases.reshape(-1), item_biases.reshape(-1),
                      global_bias)
